# speedup vs baseline: 1.4482x; 1.4482x over previous
import functools

import jax
import jax.numpy as jnp
from jax import lax
from jax.experimental import pallas as pl
from jax.experimental.pallas import tpu as pltpu
from jax.experimental.pallas import tpu_sc as plsc

_V = 1024
_D = 4096
_B = 8 * 1024


@functools.lru_cache(maxsize=None)
def _make_gather(V, D, B):
    info = plsc.get_sparse_core_info()
    NC, NS = info.num_cores, info.num_subcores
    NW = NC * NS
    assert B % (8 * NW) == 0
    b_per_w = B // NW
    C = 8
    n_chunks = b_per_w // C
    mesh = plsc.VectorSubcoreMesh(core_axis_name="c", subcore_axis_name="s")

    @functools.partial(
        pl.kernel,
        mesh=mesh,
        out_type=jax.ShapeDtypeStruct((B, D), jnp.float32),
        scratch_types=[
            pltpu.VMEM((b_per_w,), jnp.int32),
            pltpu.VMEM((C, D), jnp.float32),
            pltpu.SemaphoreType.DMA,
        ],
    )
    def k(idx_hbm, table_hbm, out_hbm, idx_v, rows_v, sem):
        wid = lax.axis_index("s") * NC + lax.axis_index("c")
        base = wid * b_per_w
        pltpu.sync_copy(idx_hbm.at[pl.ds(base, b_per_w)], idx_v)

        def body(i, carry):
            pltpu.async_copy(
                table_hbm.at[idx_v.at[pl.ds(i * C, C)]], rows_v, sem
            ).wait()
            pltpu.sync_copy(rows_v, out_hbm.at[pl.ds(base + i * C, C)])
            return carry

        lax.fori_loop(0, n_chunks, body, 0)

    return k


def kernel(indices, table):
    idx_flat = indices.reshape(-1).astype(jnp.int32)
    out = _make_gather(_V, _D, _B)(idx_flat, table)
    return out.reshape(indices.shape[0], indices.shape[1], table.shape[1])


# speedup vs baseline: 1.7306x; 1.1950x over previous
import functools

import jax
import jax.numpy as jnp
from jax import lax
from jax.experimental import pallas as pl
from jax.experimental.pallas import tpu as pltpu
from jax.experimental.pallas import tpu_sc as plsc

_V = 1024
_D = 4096
_B = 8 * 1024


@functools.lru_cache(maxsize=None)
def _make_gather(V, D, B):
    info = plsc.get_sparse_core_info()
    NC, NS = info.num_cores, info.num_subcores
    NW = NC * NS
    assert B % (8 * NW) == 0
    b_per_w = B // NW
    C = 8
    n_chunks = b_per_w // C
    mesh = plsc.VectorSubcoreMesh(core_axis_name="c", subcore_axis_name="s")

    n_pairs = n_chunks // 2
    assert n_chunks == 2 * n_pairs and n_pairs >= 2

    @functools.partial(
        pl.kernel,
        mesh=mesh,
        out_type=jax.ShapeDtypeStruct((B, D), jnp.float32),
        scratch_types=[
            pltpu.VMEM((b_per_w,), jnp.int32),
            pltpu.VMEM((C, D), jnp.float32),
            pltpu.VMEM((C, D), jnp.float32),
            pltpu.SemaphoreType.DMA,
            pltpu.SemaphoreType.DMA,
            pltpu.SemaphoreType.DMA,
            pltpu.SemaphoreType.DMA,
        ],
    )
    def k(idx_hbm, table_hbm, out_hbm, idx_v, buf0, buf1, g0, g1, o0, o1):
        wid = lax.axis_index("s") * NC + lax.axis_index("c")
        base = wid * b_per_w
        pltpu.sync_copy(idx_hbm.at[pl.ds(base, b_per_w)], idx_v)

        def gather(i, buf, sem):
            return pltpu.make_async_copy(
                table_hbm.at[idx_v.at[pl.ds(i * C, C)]], buf, sem)

        def outcopy(i, buf, sem):
            return pltpu.make_async_copy(
                buf, out_hbm.at[pl.ds(base + i * C, C)], sem)

        gather(0, buf0, g0).start()

        def body(j, carry):
            i0 = 2 * j

            @pl.when(j >= 1)
            def _():
                outcopy(i0 - 1, buf1, o1).wait()

            gather(i0 + 1, buf1, g1).start()
            gather(i0, buf0, g0).wait()
            outcopy(i0, buf0, o0).start()

            @pl.when(j + 1 < n_pairs)
            def _():
                outcopy(i0, buf0, o0).wait()
                gather(i0 + 2, buf0, g0).start()

            gather(i0 + 1, buf1, g1).wait()
            outcopy(i0 + 1, buf1, o1).start()
            return carry

        lax.fori_loop(0, n_pairs, body, 0)
        outcopy(n_chunks - 2, buf0, o0).wait()
        outcopy(n_chunks - 1, buf1, o1).wait()

    return k


def kernel(indices, table):
    idx_flat = indices.reshape(-1).astype(jnp.int32)
    out = _make_gather(_V, _D, _B)(idx_flat, table)
    return out.reshape(indices.shape[0], indices.shape[1], table.shape[1])
